# baseline (speedup 1.0000x reference)
import functools
import math

import jax
import jax.numpy as jnp
from jax import lax
from jax.experimental import pallas as pl
from jax.experimental.pallas import tpu as pltpu
from jax.experimental.pallas import tpu_sc as plsc

_NUM_CORES = 2
_NUM_SUBCORES = 16
_LANES = 16
_NUM_WORKERS = _NUM_CORES * _NUM_SUBCORES


@functools.partial(jax.jit, static_argnames=())
def _gather_scaled(idx, table):
    B = idx.shape[0]
    V, D = table.shape
    assert B % _NUM_WORKERS == 0
    b_per_w = B // _NUM_WORKERS
    ch = 64
    n_chunk = b_per_w // ch
    scale = math.sqrt(float(D))

    mesh = plsc.VectorSubcoreMesh(core_axis_name="c", subcore_axis_name="s")

    @functools.partial(
        pl.kernel,
        mesh=mesh,
        out_type=jax.ShapeDtypeStruct((B, D), jnp.float32),
        scratch_types=[
            pltpu.VMEM((b_per_w,), jnp.int32),
            pltpu.VMEM((ch, D), jnp.float32),
            pltpu.SemaphoreType.DMA,
        ],
    )
    def emb_kernel(table_hbm, idx_hbm, out_hbm, idx_v, rows_v, sem):
        wid = lax.axis_index("s") * _NUM_CORES + lax.axis_index("c")
        base = wid * b_per_w
        pltpu.sync_copy(idx_hbm.at[pl.ds(base, b_per_w)], idx_v)
        for c in range(n_chunk):
            pltpu.async_copy(
                table_hbm.at[idx_v.at[pl.ds(c * ch, ch)]], rows_v, sem
            ).wait()

            def col_body(j, _):
                r = j // (D // _LANES)
                col = (j % (D // _LANES)) * _LANES
                rows_v[r, pl.ds(col, _LANES)] = (
                    rows_v[r, pl.ds(col, _LANES)] * scale
                )
                return 0

            lax.fori_loop(0, ch * (D // _LANES), col_body, 0)
            pltpu.sync_copy(rows_v, out_hbm.at[pl.ds(base + c * ch, ch)])

    return emb_kernel(table, idx)


def kernel(x, table):
    lead_shape = x.shape
    idx = x.reshape(-1).astype(jnp.int32)
    out = _gather_scaled(idx, table)
    return out.reshape(*lead_shape, table.shape[1])


# speedup vs baseline: 2.4460x; 2.4460x over previous
import functools
import math

import jax
import jax.numpy as jnp
from jax import lax
from jax.experimental import pallas as pl
from jax.experimental.pallas import tpu as pltpu
from jax.experimental.pallas import tpu_sc as plsc

_NUM_CORES = 2
_NUM_SUBCORES = 16
_LANES = 16
_NUM_WORKERS = _NUM_CORES * _NUM_SUBCORES


@functools.partial(jax.jit, static_argnames=())
def _gather_scaled(idx, table):
    B = idx.shape[0]
    V, D = table.shape
    assert B % _NUM_WORKERS == 0
    b_per_w = B // _NUM_WORKERS
    ch = 32
    n_chunk = b_per_w // ch
    scale = math.sqrt(float(D))

    mesh = plsc.VectorSubcoreMesh(core_axis_name="c", subcore_axis_name="s")

    @functools.partial(
        pl.kernel,
        mesh=mesh,
        out_type=jax.ShapeDtypeStruct((B, D), jnp.float32),
        scratch_types=[
            pltpu.VMEM((b_per_w,), jnp.int32),
            pltpu.VMEM((ch, D), jnp.float32),
            pltpu.VMEM((ch, D), jnp.float32),
            pltpu.SemaphoreType.DMA,
            pltpu.SemaphoreType.DMA,
            pltpu.SemaphoreType.DMA,
            pltpu.SemaphoreType.DMA,
        ],
    )
    def emb_kernel(table_hbm, idx_hbm, out_hbm, idx_v, buf0, buf1,
                   g0, g1, s0, s1):
        bufs = (buf0, buf1)
        gsem = (g0, g1)
        ssem = (s0, s1)
        wid = lax.axis_index("s") * _NUM_CORES + lax.axis_index("c")
        base = wid * b_per_w
        pltpu.sync_copy(idx_hbm.at[pl.ds(base, b_per_w)], idx_v)

        def start_gather(c, b):
            return pltpu.async_copy(
                table_hbm.at[idx_v.at[pl.ds(c * ch, ch)]], bufs[b], gsem[b]
            )

        def start_scatter(c, b):
            return pltpu.async_copy(
                bufs[b], out_hbm.at[pl.ds(base + c * ch, ch)], ssem[b]
            )

        g_h = [start_gather(0, 0), None]
        s_h = [None, None]
        for c in range(n_chunk):
            b = c & 1
            if c + 1 < n_chunk:
                if s_h[1 - b] is not None:
                    s_h[1 - b].wait()
                g_h[1 - b] = start_gather(c + 1, 1 - b)
            g_h[b].wait()
            buf = bufs[b]

            def row_body(r, _, buf=buf):
                for j in range(D // _LANES):
                    buf[r, pl.ds(j * _LANES, _LANES)] = (
                        buf[r, pl.ds(j * _LANES, _LANES)] * scale
                    )
                return 0

            lax.fori_loop(0, ch, row_body, 0)
            s_h[b] = start_scatter(c, b)
        s_h[0].wait()
        s_h[1].wait()

    return emb_kernel(table, idx)


def kernel(x, table):
    lead_shape = x.shape
    idx = x.reshape(-1).astype(jnp.int32)
    out = _gather_scaled(idx, table)
    return out.reshape(*lead_shape, table.shape[1])


# speedup vs baseline: 2.6200x; 1.0712x over previous
import functools
import math

import jax
import jax.numpy as jnp
from jax import lax
from jax.experimental import pallas as pl
from jax.experimental.pallas import tpu as pltpu
from jax.experimental.pallas import tpu_sc as plsc

_NUM_CORES = 2
_NUM_SUBCORES = 16
_LANES = 16
_NUM_WORKERS = _NUM_CORES * _NUM_SUBCORES


@functools.partial(jax.jit, static_argnames=())
def _gather_scaled(idx, table):
    B = idx.shape[0]
    V, D = table.shape
    assert B % _NUM_WORKERS == 0
    b_per_w = B // _NUM_WORKERS
    ch = 32
    n_chunk = b_per_w // ch
    n_buf = 3
    scale = math.sqrt(float(D))

    mesh = plsc.VectorSubcoreMesh(core_axis_name="c", subcore_axis_name="s")

    @functools.partial(
        pl.kernel,
        mesh=mesh,
        out_type=jax.ShapeDtypeStruct((B, D), jnp.float32),
        scratch_types=[
            pltpu.VMEM((b_per_w,), jnp.int32),
        ]
        + [pltpu.VMEM((ch, D), jnp.float32)] * n_buf
        + [pltpu.SemaphoreType.DMA] * (2 * n_buf),
    )
    def emb_kernel(table_hbm, idx_hbm, out_hbm, idx_v, *bufs_and_sems):
        bufs = bufs_and_sems[:n_buf]
        gsem = bufs_and_sems[n_buf:2 * n_buf]
        ssem = bufs_and_sems[2 * n_buf:]
        wid = lax.axis_index("s") * _NUM_CORES + lax.axis_index("c")
        base = wid * b_per_w
        pltpu.sync_copy(idx_hbm.at[pl.ds(base, b_per_w)], idx_v)

        def start_gather(c, b):
            return pltpu.async_copy(
                table_hbm.at[idx_v.at[pl.ds(c * ch, ch)]], bufs[b], gsem[b]
            )

        def start_scatter(c, b):
            return pltpu.async_copy(
                bufs[b], out_hbm.at[pl.ds(base + c * ch, ch)], ssem[b]
            )

        g_h = [None] * n_buf
        s_h = [None] * n_buf
        for c in range(min(n_buf - 1, n_chunk)):
            g_h[c] = start_gather(c, c)
        for c in range(n_chunk):
            b = c % n_buf
            pre = c + n_buf - 1
            if pre < n_chunk:
                b2 = pre % n_buf
                if s_h[b2] is not None:
                    s_h[b2].wait()
                g_h[b2] = start_gather(pre, b2)
            g_h[b].wait()
            buf = bufs[b]

            @plsc.parallel_loop(0, ch)
            def _(r, buf=buf):
                for j in range(D // _LANES):
                    buf[r, pl.ds(j * _LANES, _LANES)] = (
                        buf[r, pl.ds(j * _LANES, _LANES)] * scale
                    )

            s_h[b] = start_scatter(c, b)
        for b in range(n_buf):
            if s_h[b] is not None:
                s_h[b].wait()

    return emb_kernel(table, idx)


def kernel(x, table):
    lead_shape = x.shape
    idx = x.reshape(-1).astype(jnp.int32)
    out = _gather_scaled(idx, table)
    return out.reshape(*lead_shape, table.shape[1])
